# baseline (speedup 1.0000x reference)
import functools

import jax
import jax.numpy as jnp
import numpy as np
from jax import lax
from jax.experimental import pallas as pl
from jax.experimental.pallas import tpu as pltpu
from jax.experimental.pallas import tpu_sc as plsc

B = 1024
DOMAIN_NUM = 3
MAX_LEN = 20
SEQ = DOMAIN_NUM * MAX_LEN
DIM = 64
K = 512
N_TOKENS = B * SEQ

_NC, _NS = 2, 16
NW = _NC * _NS
ROWS_PER_W = N_TOKENS // NW
CHUNK = 128
N_CHUNKS = ROWS_PER_W // CHUNK


def _sc_gather(ids_flat, table, n_tokens, chunk):
    rows_per_w = n_tokens // NW
    n_chunks = rows_per_w // chunk
    mesh = plsc.VectorSubcoreMesh(
        core_axis_name="c", subcore_axis_name="s",
        num_cores=_NC, num_subcores=_NS)

    @functools.partial(
        pl.kernel,
        out_type=jax.ShapeDtypeStruct((n_tokens, 2 * DIM), jnp.float32),
        mesh=mesh,
        scratch_types=[
            pltpu.VMEM((rows_per_w,), jnp.int32),
            pltpu.VMEM((chunk, DIM), jnp.float32),
            pltpu.VMEM((chunk, DIM), jnp.float32),
            pltpu.SemaphoreType.DMA,
            pltpu.SemaphoreType.DMA,
        ],
        compiler_params=pltpu.CompilerParams(use_tc_tiling_on_sc=False),
    )
    def gather_k(ids_hbm, table_hbm, out_hbm,
                 idx_v, rows0, rows1, sem0, sem1):
        wid = lax.axis_index("s") * _NC + lax.axis_index("c")
        base = wid * rows_per_w
        pltpu.sync_copy(ids_hbm.at[pl.ds(base, rows_per_w)], idx_v)
        bufs = (rows0, rows1)
        sems = (sem0, sem1)
        cp = pltpu.async_copy(
            table_hbm.at[idx_v.at[pl.ds(0, chunk)]], bufs[0], sems[0])
        for i in range(n_chunks):
            nxt = None
            if i + 1 < n_chunks:
                nxt = pltpu.async_copy(
                    table_hbm.at[idx_v.at[pl.ds((i + 1) * chunk, chunk)]],
                    bufs[(i + 1) % 2], sems[(i + 1) % 2])
            cp.wait()
            pltpu.sync_copy(bufs[i % 2],
                            out_hbm.at[pl.ds(base + i * chunk, chunk),
                                       pl.ds(0, DIM)])
            cp = nxt

    return gather_k(ids_flat, table)


BB = 64
TB = BB * SEQ

_S_BLOCK = np.kron(np.eye(BB, dtype=np.float32),
                   np.ones((1, SEQ), dtype=np.float32))
_LANE_MASK = np.concatenate(
    [np.ones((1, DIM), np.float32), np.zeros((1, DIM), np.float32)], axis=1)


def _tc_body(x_ref, m_ref, cb_ref, cbw_ref, lm_ref, s_ref, o_ref):
    x2r = x_ref[...]
    cb = cb_ref[...]
    cbw = cbw_ref[...]
    S = s_ref[...]
    maskv = m_ref[0]
    x2 = jnp.where(lm_ref[...] > 0.5, x2r, 0.0)
    cnorm = lax.dot_general(
        jnp.ones((1, DIM), jnp.float32), cb * cb,
        (((1,), (1,)), ((), ())),
        preferred_element_type=jnp.float32)
    scores = cnorm - 2.0 * lax.dot_general(
        x2, cbw, (((1,), (1,)), ((), ())),
        preferred_element_type=jnp.float32)
    minv = jnp.min(scores, axis=1, keepdims=True)
    onehot = (scores <= minv).astype(jnp.float32)
    counts = jnp.dot(S, onehot,
                     preferred_element_type=jnp.float32)
    vq_sum = jnp.dot(counts, cb,
                     preferred_element_type=jnp.float32)
    M = S * maskv
    xm2 = jnp.dot(M, x2, preferred_element_type=jnp.float32)
    denom = jnp.maximum(jnp.sum(M, axis=1, keepdims=True), 1.0)
    o_ref[:, :DIM] = vq_sum / denom
    o_ref[:, DIM:] = xm2[:, :DIM] / denom


def _tc_vq(x2, mask_flat, code_book, cbw, interpret=False):
    grid = x2.shape[0] // TB
    return pl.pallas_call(
        _tc_body,
        grid=(grid,),
        in_specs=[
            pl.BlockSpec((TB, 2 * DIM), lambda i: (i, 0)),
            pl.BlockSpec((1, 1, TB), lambda i: (i, 0, 0)),
            pl.BlockSpec((K, DIM), lambda i: (0, 0)),
            pl.BlockSpec((K, 2 * DIM), lambda i: (0, 0)),
            pl.BlockSpec((1, 2 * DIM), lambda i: (0, 0)),
            pl.BlockSpec((BB, TB), lambda i: (0, 0)),
        ],
        out_specs=pl.BlockSpec((BB, 2 * DIM), lambda i: (i, 0)),
        out_shape=jax.ShapeDtypeStruct((grid * BB, 2 * DIM), jnp.float32),
        interpret=interpret,
    )(x2, mask_flat, code_book, cbw,
      jnp.asarray(_LANE_MASK), jnp.asarray(_S_BLOCK))


N_WAVES = 4
_WTOK = N_TOKENS // N_WAVES
_WCHUNK = 80


def kernel(history_item_ids, history_item_masks, embedding_table, code_book):
    ids_flat = history_item_ids.reshape(N_TOKENS).astype(jnp.int32)
    mask_flat = history_item_masks.reshape(B // BB, 1, TB).astype(jnp.float32)
    cbw = jnp.concatenate(
        [code_book, jnp.zeros((K, DIM), jnp.float32)], axis=1)
    mb = (B // BB) // N_WAVES
    outs = []
    for w in range(N_WAVES):
        xw = _sc_gather(ids_flat[w * _WTOK:(w + 1) * _WTOK],
                        embedding_table, _WTOK, _WCHUNK)
        outs.append(_tc_vq(xw, mask_flat[w * mb:(w + 1) * mb],
                           code_book, cbw))
    return jnp.concatenate(outs, axis=0)


# speedup vs baseline: 1.0537x; 1.0537x over previous
import functools

import jax
import jax.numpy as jnp
import numpy as np
from jax import lax
from jax.experimental import pallas as pl
from jax.experimental.pallas import tpu as pltpu
from jax.experimental.pallas import tpu_sc as plsc

B = 1024
DOMAIN_NUM = 3
MAX_LEN = 20
SEQ = DOMAIN_NUM * MAX_LEN
DIM = 64
K = 512
N_TOKENS = B * SEQ

_NC, _NS = 2, 16
NW = _NC * _NS
ROWS_PER_W = N_TOKENS // NW
CHUNK = 128
N_CHUNKS = ROWS_PER_W // CHUNK


def _sc_gather(ids_flat, table, n_tokens, chunk):
    rows_per_w = n_tokens // NW
    n_chunks = rows_per_w // chunk
    mesh = plsc.VectorSubcoreMesh(
        core_axis_name="c", subcore_axis_name="s",
        num_cores=_NC, num_subcores=_NS)

    @functools.partial(
        pl.kernel,
        out_type=jax.ShapeDtypeStruct((n_tokens, 2 * DIM), jnp.float32),
        mesh=mesh,
        scratch_types=[
            pltpu.VMEM((rows_per_w,), jnp.int32),
            pltpu.VMEM((chunk, DIM), jnp.float32),
            pltpu.VMEM((chunk, DIM), jnp.float32),
            pltpu.SemaphoreType.DMA,
            pltpu.SemaphoreType.DMA,
        ],
        compiler_params=pltpu.CompilerParams(use_tc_tiling_on_sc=False),
    )
    def gather_k(ids_hbm, table_hbm, out_hbm,
                 idx_v, rows0, rows1, sem0, sem1):
        wid = lax.axis_index("s") * _NC + lax.axis_index("c")
        base = wid * rows_per_w
        pltpu.sync_copy(ids_hbm.at[pl.ds(base, rows_per_w)], idx_v)
        bufs = (rows0, rows1)
        sems = (sem0, sem1)
        cp = pltpu.async_copy(
            table_hbm.at[idx_v.at[pl.ds(0, chunk)]], bufs[0], sems[0])
        for i in range(n_chunks):
            nxt = None
            if i + 1 < n_chunks:
                nxt = pltpu.async_copy(
                    table_hbm.at[idx_v.at[pl.ds((i + 1) * chunk, chunk)]],
                    bufs[(i + 1) % 2], sems[(i + 1) % 2])
            cp.wait()
            pltpu.sync_copy(bufs[i % 2],
                            out_hbm.at[pl.ds(base + i * chunk, chunk),
                                       pl.ds(0, DIM)])
            cp = nxt

    return gather_k(ids_flat, table)


BB = 64
TB = BB * SEQ

_S_BLOCK = np.kron(np.eye(BB, dtype=np.float32),
                   np.ones((1, SEQ), dtype=np.float32))
_LANE_MASK = np.concatenate(
    [np.ones((1, DIM), np.float32), np.zeros((1, DIM), np.float32)], axis=1)


def _tc_body(x_ref, m_ref, cb_ref, cbw_ref, lm_ref, s_ref, o_ref):
    x2r = x_ref[...]
    cb = cb_ref[...]
    cbw = cbw_ref[...]
    S = s_ref[...]
    maskv = m_ref[0]
    x2 = jnp.where(lm_ref[...] > 0.5, x2r, 0.0)
    cnorm = lax.dot_general(
        jnp.ones((1, DIM), jnp.float32), cb * cb,
        (((1,), (1,)), ((), ())),
        preferred_element_type=jnp.float32)
    scores = cnorm - 2.0 * lax.dot_general(
        x2, cbw, (((1,), (1,)), ((), ())),
        preferred_element_type=jnp.float32)
    minv = jnp.min(scores, axis=1, keepdims=True)
    onehot = (scores <= minv).astype(jnp.float32)
    counts = jnp.dot(S, onehot,
                     preferred_element_type=jnp.float32)
    vq_sum = jnp.dot(counts, cb,
                     preferred_element_type=jnp.float32)
    M = S * maskv
    xm2 = jnp.dot(M, x2, preferred_element_type=jnp.float32)
    denom = jnp.maximum(jnp.sum(M, axis=1, keepdims=True), 1.0)
    o_ref[:, :DIM] = vq_sum / denom
    o_ref[:, DIM:] = xm2[:, :DIM] / denom


def _tc_vq(x2, mask_flat, code_book, cbw, interpret=False):
    grid = x2.shape[0] // TB
    return pl.pallas_call(
        _tc_body,
        grid=(grid,),
        in_specs=[
            pl.BlockSpec((TB, 2 * DIM), lambda i: (i, 0)),
            pl.BlockSpec((1, 1, TB), lambda i: (i, 0, 0)),
            pl.BlockSpec((K, DIM), lambda i: (0, 0)),
            pl.BlockSpec((K, 2 * DIM), lambda i: (0, 0)),
            pl.BlockSpec((1, 2 * DIM), lambda i: (0, 0)),
            pl.BlockSpec((BB, TB), lambda i: (0, 0)),
        ],
        out_specs=pl.BlockSpec((BB, 2 * DIM), lambda i: (i, 0)),
        out_shape=jax.ShapeDtypeStruct((grid * BB, 2 * DIM), jnp.float32),
        interpret=interpret,
    )(x2, mask_flat, code_book, cbw,
      jnp.asarray(_LANE_MASK), jnp.asarray(_S_BLOCK))


N_WAVES = 2
_WTOK = N_TOKENS // N_WAVES
_WCHUNK = _WTOK // NW // 8


def kernel(history_item_ids, history_item_masks, embedding_table, code_book):
    ids_flat = history_item_ids.reshape(N_TOKENS).astype(jnp.int32)
    mask_flat = history_item_masks.reshape(B // BB, 1, TB).astype(jnp.float32)
    cbw = jnp.concatenate(
        [code_book, jnp.zeros((K, DIM), jnp.float32)], axis=1)
    mb = (B // BB) // N_WAVES
    outs = []
    for w in range(N_WAVES):
        xw = _sc_gather(ids_flat[w * _WTOK:(w + 1) * _WTOK],
                        embedding_table, _WTOK, _WCHUNK)
        outs.append(_tc_vq(xw, mask_flat[w * mb:(w + 1) * mb],
                           code_book, cbw))
    return jnp.concatenate(outs, axis=0)
